# Initial kernel scaffold; baseline (speedup 1.0000x reference)
import jax
import jax.numpy as jnp
from jax.experimental import pallas as pl


def kernel(x, Wg, W1, b1, W2, b2):
    raise NotImplementedError("write your pallas kernel here")


# speedup vs baseline: 1.3884x; 1.3884x over previous
import functools

import jax
import jax.numpy as jnp
from jax.experimental import pallas as pl
from jax.experimental.pallas import tpu as pltpu

_LANES = 128


def _router_body(x_ref, wg_ref, comb_ref, aux_ref):
    x = x_ref[...]
    wg = wg_ref[...]
    logits = jnp.dot(x, wg, preferred_element_type=jnp.float32)
    S = x.shape[0]
    lane = jax.lax.broadcasted_iota(jnp.int32, (S, _LANES), 1)
    E = 8
    neg = jnp.full_like(logits, -jnp.inf)
    logits = jnp.where(lane < E, logits, neg)
    m = jnp.max(logits, axis=1, keepdims=True)
    ex = jnp.exp(logits - m)
    probs = ex / jnp.sum(ex, axis=1, keepdims=True)

    m1 = jnp.max(probs, axis=1, keepdims=True)
    big = jnp.int32(10 ** 9)
    i1 = jnp.min(jnp.where(probs == m1, lane, big), axis=1, keepdims=True)
    probs_m = jnp.where(lane == i1, -1.0, probs)
    m2 = jnp.max(probs_m, axis=1, keepdims=True)
    i2 = jnp.min(jnp.where(probs_m == m2, lane, big), axis=1, keepdims=True)

    denom = m1 + m2
    w1 = m1 / denom
    w2 = m2 / denom
    oh1 = (lane == i1).astype(jnp.float32)
    oh2 = (lane == i2).astype(jnp.float32)
    comb = w1 * oh1 + w2 * oh2
    comb_ref[...] = comb

    counts = jnp.sum(oh1 + oh2, axis=0, keepdims=True)
    avg_prob = jnp.mean(probs, axis=0, keepdims=True)
    aux_ref[0, 0] = jnp.float32(E) * jnp.sum(counts * avg_prob)


def _ffn_body(comb_ref, x_ref, w1_ref, b1_ref, w2_ref, b2_ref, out_ref,
              acc_ref, *, n_dt):
    e = pl.program_id(0)
    dt = pl.program_id(1)
    st = pl.program_id(2)
    ts = x_ref.shape[0]
    x = x_ref[...]
    w1 = w1_ref[0]
    b1 = b1_ref[0]
    w2 = w2_ref[0]
    b2 = b2_ref[0]

    h = jnp.dot(x, w1, preferred_element_type=jnp.float32) + b1
    h = h * 0.5 * (1.0 + jax.lax.erf(h * 0.7071067811865476))
    part = jnp.dot(h, w2, preferred_element_type=jnp.float32)
    part = jnp.where(dt == 0, part + b2, part)

    lane = jax.lax.broadcasted_iota(jnp.int32, (1, _LANES), 1)
    oh_e = (lane == e).astype(jnp.float32)
    c = jnp.sum(comb_ref[...] * oh_e, axis=1, keepdims=True)
    contrib = c * part

    first = jnp.logical_and(e == 0, dt == 0)
    rows = pl.ds(st * ts, ts)

    @pl.when(first)
    def _():
        acc_ref[rows, :] = contrib

    @pl.when(jnp.logical_not(first))
    def _():
        acc_ref[rows, :] = acc_ref[rows, :] + contrib

    out_ref[...] = acc_ref[rows, :]


def kernel(x, Wg, W1, b1, W2, b2):
    B, S, H = x.shape
    E, _, D = W1.shape
    x2 = x.reshape(S, H)

    wg_pad = jnp.zeros((H, _LANES), jnp.float32).at[:, :E].set(Wg)

    comb, aux = pl.pallas_call(
        _router_body,
        out_shape=(
            jax.ShapeDtypeStruct((S, _LANES), jnp.float32),
            jax.ShapeDtypeStruct((1, 1), jnp.float32),
        ),
        in_specs=[
            pl.BlockSpec(memory_space=pltpu.VMEM),
            pl.BlockSpec(memory_space=pltpu.VMEM),
        ],
        out_specs=(
            pl.BlockSpec(memory_space=pltpu.VMEM),
            pl.BlockSpec(memory_space=pltpu.SMEM),
        ),
    )(x2, wg_pad)

    TS = 128
    DB = 1024
    n_st = S // TS
    n_dt = D // DB

    out = pl.pallas_call(
        functools.partial(_ffn_body, n_dt=n_dt),
        grid=(E, n_dt, n_st),
        in_specs=[
            pl.BlockSpec((TS, _LANES), lambda e, dt, st: (st, 0)),
            pl.BlockSpec((TS, H), lambda e, dt, st: (st, 0)),
            pl.BlockSpec((1, H, DB), lambda e, dt, st: (e, 0, dt)),
            pl.BlockSpec((1, 1, DB), lambda e, dt, st: (e, 0, dt)),
            pl.BlockSpec((1, DB, H), lambda e, dt, st: (e, dt, 0)),
            pl.BlockSpec((1, 1, H), lambda e, dt, st: (e, 0, 0)),
        ],
        out_specs=pl.BlockSpec((TS, H), lambda e, dt, st: (st, 0)),
        out_shape=jax.ShapeDtypeStruct((S, H), jnp.float32),
        scratch_shapes=[pltpu.VMEM((S, H), jnp.float32)],
    )(comb, x2, W1, b1.reshape(E, 1, D), W2, b2.reshape(E, 1, H))

    return out.reshape(B, S, H), aux[0, 0]
